# baseline (speedup 1.0000x reference)
import functools

import jax
import jax.numpy as jnp
from jax import lax
from jax.experimental import pallas as pl
from jax.experimental.pallas import tpu as pltpu
from jax.experimental.pallas import tpu_sc as plsc

_S = 30.0
_M = 0.35
_SUB = 8
_C1 = _S * 1.4426950408889634
_NW = 32


def _sc_gather_build(batch):
    b_per_w = batch // _NW
    mesh = plsc.VectorSubcoreMesh(core_axis_name="c", subcore_axis_name="s")

    @functools.partial(
        pl.kernel, mesh=mesh,
        out_type=jax.ShapeDtypeStruct((batch, batch), jnp.float32),
        scratch_types=[
            pltpu.VMEM((b_per_w,), jnp.int32),
            pltpu.VMEM((b_per_w, batch), jnp.float32),
            pltpu.SemaphoreType.DMA,
        ],
    )
    def gather_k(xt_hbm, lbl_hbm, out_hbm, idx_v, rows_v, sem):
        wid = lax.axis_index("s") * 2 + lax.axis_index("c")
        base = wid * b_per_w
        pltpu.sync_copy(lbl_hbm.at[pl.ds(base, b_per_w)], idx_v)
        pltpu.async_copy(xt_hbm.at[idx_v], rows_v, sem).wait()
        pltpu.sync_copy(rows_v, out_hbm.at[pl.ds(base, b_per_w)])

    return gather_k


def _stripe_body(n_rows, n_cls, n_blocks, bs, xt_ref, g_ref, out_ref,
                 m_ref, s_ref):
    i = pl.program_id(0)
    ns = bs // _SUB
    ns_tail = (n_cls - (n_blocks - 1) * bs) // _SUB

    @pl.when(i == 0)
    def _init():
        m_ref[...] = jnp.full_like(m_ref, -jnp.inf)
        s_ref[...] = jnp.zeros_like(s_ref)

    def update(n_slices):
        m_old = m_ref[...]
        bm = m_old
        for k in range(n_slices):
            bm = jnp.maximum(bm, xt_ref[k * _SUB:(k + 1) * _SUB, :])
        acc = s_ref[...] * jnp.exp2(_C1 * (m_old - bm))
        for k in range(n_slices):
            ch = xt_ref[k * _SUB:(k + 1) * _SUB, :]
            acc = acc + jnp.exp2(_C1 * (ch - bm))
        s_ref[...] = acc
        m_ref[...] = bm

    @pl.when(i < n_blocks - 1)
    def _main():
        update(ns)

    @pl.when(i == n_blocks - 1)
    def _tail():
        update(ns_tail)

        m8 = m_ref[...]
        mrow = jnp.max(m8, axis=0, keepdims=True)
        srow = jnp.sum(s_ref[...] * jnp.exp2(_C1 * (m8 - mrow)),
                       axis=0, keepdims=True)
        g = g_ref[...]
        rid = lax.broadcasted_iota(jnp.int32, g.shape, 0)
        cid = lax.broadcasted_iota(jnp.int32, g.shape, 1)
        t = jnp.sum(jnp.where(rid == cid, g, 0.0), axis=0, keepdims=True)
        e1 = jnp.exp(_S * (t - mrow))
        e2 = jnp.exp(_S * (t - _M - mrow))
        s_corr = jnp.maximum(srow - e1, 0.0) + e2
        nll = jnp.log(s_corr) + _S * mrow - _S * (t - _M)
        out_ref[...] = jnp.sum(nll, axis=(0, 1), keepdims=True) / n_rows


@jax.jit
def kernel(input, label):
    n_rows, n_cls = input.shape
    xt = input.T
    lbl = label.astype(jnp.int32)

    g = _sc_gather_build(n_rows)(xt, lbl)

    bs = 2048
    n_blocks = pl.cdiv(n_cls, bs)
    body = lambda *refs: _stripe_body(n_rows, n_cls, n_blocks, bs, *refs)
    out = pl.pallas_call(
        body,
        grid=(n_blocks,),
        in_specs=[
            pl.BlockSpec((bs, n_rows), lambda i: (i, 0)),
            pl.BlockSpec((n_rows, n_rows), lambda i: (0, 0)),
        ],
        out_specs=pl.BlockSpec((1, 1), lambda i: (0, 0)),
        out_shape=jax.ShapeDtypeStruct((1, 1), jnp.float32),
        scratch_shapes=[
            pltpu.VMEM((_SUB, n_rows), jnp.float32),
            pltpu.VMEM((_SUB, n_rows), jnp.float32),
        ],
    )(xt, g)
    return out[0, 0]


# speedup vs baseline: 1.0205x; 1.0205x over previous
import functools

import jax
import jax.numpy as jnp
from jax import lax
from jax.experimental import pallas as pl
from jax.experimental.pallas import tpu as pltpu
from jax.experimental.pallas import tpu_sc as plsc

_S = 30.0
_M = 0.35
_SUB = 8
_C1 = _S * 1.4426950408889634
_NW = 32


def _sc_gather_build(batch):
    b_per_w = batch // _NW
    mesh = plsc.VectorSubcoreMesh(core_axis_name="c", subcore_axis_name="s")

    @functools.partial(
        pl.kernel, mesh=mesh,
        out_type=jax.ShapeDtypeStruct((batch, batch), jnp.float32),
        scratch_types=[
            pltpu.VMEM((b_per_w,), jnp.int32),
            pltpu.VMEM((b_per_w, batch), jnp.float32),
            pltpu.SemaphoreType.DMA,
        ],
    )
    def gather_k(xt_hbm, lbl_hbm, out_hbm, idx_v, rows_v, sem):
        wid = lax.axis_index("s") * 2 + lax.axis_index("c")
        base = wid * b_per_w
        pltpu.sync_copy(lbl_hbm.at[pl.ds(base, b_per_w)], idx_v)
        pltpu.async_copy(xt_hbm.at[idx_v], rows_v, sem).wait()
        pltpu.sync_copy(rows_v, out_hbm.at[pl.ds(base, b_per_w)])

    return gather_k


def _stripe_body(n_cls, n_blocks, bs, xt_ref, m_ref, s_ref):
    i = pl.program_id(0)
    ns = bs // _SUB
    ns_tail = (n_cls - (n_blocks - 1) * bs) // _SUB

    @pl.when(i == 0)
    def _init():
        m_ref[...] = jnp.full_like(m_ref, -jnp.inf)
        s_ref[...] = jnp.zeros_like(s_ref)

    def update(n_slices):
        m_old = m_ref[...]
        bm = m_old
        for k in range(n_slices):
            bm = jnp.maximum(bm, xt_ref[k * _SUB:(k + 1) * _SUB, :])
        acc = s_ref[...] * jnp.exp2(_C1 * (m_old - bm))
        for k in range(n_slices):
            ch = xt_ref[k * _SUB:(k + 1) * _SUB, :]
            acc = acc + jnp.exp2(_C1 * (ch - bm))
        s_ref[...] = acc
        m_ref[...] = bm

    @pl.when(i < n_blocks - 1)
    def _main():
        update(ns)

    @pl.when(i == n_blocks - 1)
    def _tail():
        update(ns_tail)


def _combine_body(n_rows, m_ref, s_ref, g_ref, out_ref):
    m8 = m_ref[...]
    mrow = jnp.max(m8, axis=0, keepdims=True)
    srow = jnp.sum(s_ref[...] * jnp.exp2(_C1 * (m8 - mrow)),
                   axis=0, keepdims=True)
    g = g_ref[...]
    rid = lax.broadcasted_iota(jnp.int32, g.shape, 0)
    cid = lax.broadcasted_iota(jnp.int32, g.shape, 1)
    t = jnp.sum(jnp.where(rid == cid, g, 0.0), axis=0, keepdims=True)
    e1 = jnp.exp(_S * (t - mrow))
    e2 = jnp.exp(_S * (t - _M - mrow))
    s_corr = jnp.maximum(srow - e1, 0.0) + e2
    nll = jnp.log(s_corr) + _S * mrow - _S * (t - _M)
    out_ref[...] = jnp.sum(nll, axis=(0, 1), keepdims=True) / n_rows


@jax.jit
def kernel(input, label):
    n_rows, n_cls = input.shape
    xt = input.T
    lbl = label.astype(jnp.int32)

    g = _sc_gather_build(n_rows)(xt, lbl)

    bs = 2048
    n_blocks = pl.cdiv(n_cls, bs)
    body = lambda *refs: _stripe_body(n_cls, n_blocks, bs, *refs)
    m8, s8 = pl.pallas_call(
        body,
        grid=(n_blocks,),
        in_specs=[pl.BlockSpec((bs, n_rows), lambda i: (i, 0))],
        out_specs=[
            pl.BlockSpec((_SUB, n_rows), lambda i: (0, 0)),
            pl.BlockSpec((_SUB, n_rows), lambda i: (0, 0)),
        ],
        out_shape=[
            jax.ShapeDtypeStruct((_SUB, n_rows), jnp.float32),
            jax.ShapeDtypeStruct((_SUB, n_rows), jnp.float32),
        ],
    )(xt)

    out = pl.pallas_call(
        functools.partial(_combine_body, n_rows),
        out_shape=jax.ShapeDtypeStruct((1, 1), jnp.float32),
    )(m8, s8, g)
    return out[0, 0]
